# baseline (speedup 1.0000x reference)
import jax
import jax.numpy as jnp
from jax import lax
from jax.experimental import pallas as pl
from jax.experimental.pallas import tpu as pltpu
from jax.experimental.pallas import tpu_sc as plsc

VOCAB = 1000000
L = 16
NC = 2
NS = 16
NW = NC * NS
B = 4096
SEQ = 200
NFULL = SEQ // L
TAIL = SEQ - NFULL * L
E = 32
WIDE = 128
NCLS = WIDE // E
VROWS = VOCAB * E // WIDE
ROWS_PER_W = B // NW
G = 16
NBLK = ROWS_PER_W // G
IDX_PER_G = G * SEQ
CHUNK = 128
NCHUNK = IDX_PER_G // CHUNK
SENT = 0x7FFFFFFF


def _sc_body(x_hbm, tbl_hbm, t0b_hbm, wb_hbm, bb_hbm,
             logits_hbm, feat_hbm,
             idx_v, rows_v, t0b_v, wb_v, bb_v,
             t_v, cnt_v, feat_v, logit_v, sem):
    cid = lax.axis_index("c")
    sid = lax.axis_index("s")
    wid = sid * NC + cid
    base_row = wid * ROWS_PER_W

    pltpu.sync_copy(t0b_hbm, t0b_v)
    pltpu.sync_copy(wb_hbm, wb_v)
    pltpu.sync_copy(bb_hbm, bb_v)
    zero = jnp.zeros((L,), jnp.float32)
    lane = lax.iota(jnp.int32, L)

    def do_block(blk, _):
        flat0 = base_row * SEQ + blk * IDX_PER_G
        pltpu.sync_copy(x_hbm.at[pl.ds(flat0, IDX_PER_G)],
                        idx_v.at[pl.ds(0, IDX_PER_G)])

        cps = []
        for j in range(NCHUNK):
            cps.append(pltpu.async_copy(
                tbl_hbm.at[idx_v.at[pl.ds(j * CHUNK, CHUNK)]],
                rows_v.at[pl.ds(j * CHUNK, CHUNK)], sem))
        for cp in cps:
            cp.wait()

        for r in range(G):
            rb = r * SEQ

            def body(i, carry):
                a0, a1, b0, b1, c0, c1, d0, d1, cnt = carry
                off = rb + i * L
                cvec = idx_v[pl.ds(off, L)]
                cnt = cnt + jnp.where(cvec == 0, 1.0, 0.0)
                for t in range(0, L, 4):
                    a0 = a0 + rows_v[off + t, pl.ds(0, L)]
                    a1 = a1 + rows_v[off + t, pl.ds(L, L)]
                    b0 = b0 + rows_v[off + t + 1, pl.ds(0, L)]
                    b1 = b1 + rows_v[off + t + 1, pl.ds(L, L)]
                    c0 = c0 + rows_v[off + t + 2, pl.ds(0, L)]
                    c1 = c1 + rows_v[off + t + 2, pl.ds(L, L)]
                    d0 = d0 + rows_v[off + t + 3, pl.ds(0, L)]
                    d1 = d1 + rows_v[off + t + 3, pl.ds(L, L)]
                return a0, a1, b0, b1, c0, c1, d0, d1, cnt

            a0, a1, b0, b1, c0, c1, d0, d1, cnt = lax.fori_loop(
                0, NFULL, body, (zero,) * 9)
            toff = rb + NFULL * L
            cvec = idx_v[pl.ds(toff, L)]
            cnt = cnt + jnp.where((lane < TAIL) & (cvec == 0), 1.0, 0.0)
            for t in range(0, TAIL, 4):
                a0 = a0 + rows_v[toff + t, pl.ds(0, L)]
                a1 = a1 + rows_v[toff + t, pl.ds(L, L)]
                b0 = b0 + rows_v[toff + t + 1, pl.ds(0, L)]
                b1 = b1 + rows_v[toff + t + 1, pl.ds(L, L)]
                c0 = c0 + rows_v[toff + t + 2, pl.ds(0, L)]
                c1 = c1 + rows_v[toff + t + 2, pl.ds(L, L)]
                d0 = d0 + rows_v[toff + t + 3, pl.ds(0, L)]
                d1 = d1 + rows_v[toff + t + 3, pl.ds(L, L)]
            plsc.store_scatter(t_v, [lane * L + r], (a0 + b0) + (c0 + d0))
            plsc.store_scatter(
                t_v, [lane * L + (G * L + r)], (a1 + b1) + (c1 + d1))
            plsc.store_scatter(cnt_v, [lane * L + r], cnt)

        n0f = cnt_v[pl.ds(0, L)]
        for k in range(1, L):
            n0f = n0f + cnt_v[pl.ds(k * L, L)]
        rdenom = 1.0 / jnp.maximum(jnp.float32(SEQ) - n0f, 1.0)
        lg = bb_v[...]
        for e in range(E):
            fe = (t_v[pl.ds(e * L, L)]
                  - n0f * t0b_v[pl.ds(e * L, L)]) * rdenom
            lg = lg + fe * wb_v[pl.ds(e * L, L)]
            plsc.store_scatter(feat_v, [lane * E + e], fe)
        logit_v[pl.ds(blk * L, L)] = lg
        pltpu.sync_copy(
            feat_v, feat_hbm.at[pl.ds((base_row + blk * G) * E, G * E)])
        return 0

    lax.fori_loop(0, NBLK, do_block, 0)
    pltpu.sync_copy(logit_v,
                    logits_hbm.at[pl.ds(base_row, ROWS_PER_W)])


@jax.jit
def _run(x_flat, tbl_wide, t0b, wb, bb):
    mesh = plsc.VectorSubcoreMesh(core_axis_name="c", subcore_axis_name="s")
    fn = pl.kernel(
        _sc_body,
        mesh=mesh,
        compiler_params=pltpu.CompilerParams(
            needs_layout_passes=False, use_tc_tiling_on_sc=False),
        out_type=[
            jax.ShapeDtypeStruct((B,), jnp.float32),
            jax.ShapeDtypeStruct((B * E,), jnp.float32),
        ],
        scratch_types=[
            pltpu.VMEM((IDX_PER_G + L,), jnp.int32),
            pltpu.VMEM((IDX_PER_G, E), jnp.float32),
            pltpu.VMEM((E * L,), jnp.float32),
            pltpu.VMEM((E * L,), jnp.float32),
            pltpu.VMEM((L,), jnp.float32),
            pltpu.VMEM((E * L,), jnp.float32),
            pltpu.VMEM((L * L,), jnp.float32),
            pltpu.VMEM((G * E,), jnp.float32),
            pltpu.VMEM((ROWS_PER_W,), jnp.float32),
            pltpu.SemaphoreType.DMA,
        ],
    )
    logits_flat, feat_flat = fn(x_flat, tbl_wide, t0b, wb, bb)
    return logits_flat.reshape(B, 1), feat_flat.reshape(B, E)


def kernel(x, embed_table, W, b):
    x_flat = jnp.asarray(x).astype(jnp.int32).reshape(-1)
    tbl_wide = embed_table
    t0b = jnp.broadcast_to(embed_table[0][:, None], (E, L)).reshape(-1)
    wb = jnp.broadcast_to(
        W.astype(jnp.float32).reshape(E, 1), (E, L)).reshape(-1)
    bb = jnp.broadcast_to(b.astype(jnp.float32), (L,))
    return _run(x_flat, tbl_wide, t0b, wb, bb)


# speedup vs baseline: 1.0524x; 1.0524x over previous
import jax
import jax.numpy as jnp
from jax import lax
from jax.experimental import pallas as pl
from jax.experimental.pallas import tpu as pltpu
from jax.experimental.pallas import tpu_sc as plsc

L = 16
NC = 2
NS = 16
NW = NC * NS
B = 4096
SEQ = 200
NFULL = SEQ // L
TAIL = SEQ - NFULL * L
E = 32
ROWS_PER_W = B // NW
G = 8
NBLK = ROWS_PER_W // G
NPAIR = NBLK // 2
IDX_PER_G = G * SEQ
CHUNK = 128
_sizes = [CHUNK] * (IDX_PER_G // CHUNK) + (
    [IDX_PER_G % CHUNK] if IDX_PER_G % CHUNK else [])
CHUNKS = [(o, s) for o, s in zip(
    [sum(_sizes[:i]) for i in range(len(_sizes))], _sizes)]


def _sc_body(x_hbm, tbl_hbm, const_hbm,
             logits_hbm, feat_hbm,
             idx_a, idx_b, rows_a, rows_b, const_v,
             t_v, cnt_v, feat_v, logit_v, sem_a, sem_b):
    cid = lax.axis_index("c")
    sid = lax.axis_index("s")
    wid = sid * NC + cid
    base_row = wid * ROWS_PER_W

    pltpu.sync_copy(const_hbm, const_v)
    zero = jnp.zeros((L,), jnp.float32)
    lane = lax.iota(jnp.int32, L)

    def load_block(blk, idx_v, rows_v, sem):
        flat0 = base_row * SEQ + blk * IDX_PER_G
        pltpu.sync_copy(x_hbm.at[pl.ds(flat0, IDX_PER_G)],
                        idx_v.at[pl.ds(0, IDX_PER_G)])
        for off, sz in CHUNKS:
            pltpu.async_copy(
                tbl_hbm.at[idx_v.at[pl.ds(off, sz)]],
                rows_v.at[pl.ds(off, sz)], sem)

    def drain(rows_v, sem):
        pltpu.make_async_copy(
            tbl_hbm.at[pl.ds(0, IDX_PER_G)], rows_v, sem).wait()

    def dim_phase(idx_v, rows_v, half):
        for r in range(G):
            rp = half * G + r
            rb = r * SEQ

            def body(i, carry):
                a0, a1, b0, b1, c0, c1, d0, d1, cnt = carry
                off = rb + i * L
                cvec = idx_v[pl.ds(off, L)]
                cnt = cnt + jnp.where(cvec == 0, 1.0, 0.0)
                for t in range(0, L, 4):
                    a0 = a0 + rows_v[off + t, pl.ds(0, L)]
                    a1 = a1 + rows_v[off + t, pl.ds(L, L)]
                    b0 = b0 + rows_v[off + t + 1, pl.ds(0, L)]
                    b1 = b1 + rows_v[off + t + 1, pl.ds(L, L)]
                    c0 = c0 + rows_v[off + t + 2, pl.ds(0, L)]
                    c1 = c1 + rows_v[off + t + 2, pl.ds(L, L)]
                    d0 = d0 + rows_v[off + t + 3, pl.ds(0, L)]
                    d1 = d1 + rows_v[off + t + 3, pl.ds(L, L)]
                return a0, a1, b0, b1, c0, c1, d0, d1, cnt

            a0, a1, b0, b1, c0, c1, d0, d1, cnt = lax.fori_loop(
                0, NFULL, body, (zero,) * 9)
            toff = rb + NFULL * L
            cvec = idx_v[pl.ds(toff, L)]
            cnt = cnt + jnp.where((lane < TAIL) & (cvec == 0), 1.0, 0.0)
            for t in range(0, TAIL, 4):
                a0 = a0 + rows_v[toff + t, pl.ds(0, L)]
                a1 = a1 + rows_v[toff + t, pl.ds(L, L)]
                b0 = b0 + rows_v[toff + t + 1, pl.ds(0, L)]
                b1 = b1 + rows_v[toff + t + 1, pl.ds(L, L)]
                c0 = c0 + rows_v[toff + t + 2, pl.ds(0, L)]
                c1 = c1 + rows_v[toff + t + 2, pl.ds(L, L)]
                d0 = d0 + rows_v[toff + t + 3, pl.ds(0, L)]
                d1 = d1 + rows_v[toff + t + 3, pl.ds(L, L)]
            plsc.store_scatter(t_v, [lane * L + rp], (a0 + b0) + (c0 + d0))
            plsc.store_scatter(
                t_v, [lane * L + (2 * G * L + rp)], (a1 + b1) + (c1 + d1))
            plsc.store_scatter(cnt_v, [lane * L + rp], cnt)

    def row_phase(p):
        pair_row0 = base_row + 2 * p * G
        n0f = cnt_v[pl.ds(0, L)]
        for k in range(1, L):
            n0f = n0f + cnt_v[pl.ds(k * L, L)]
        rdenom = 1.0 / jnp.maximum(jnp.float32(SEQ) - n0f, 1.0)
        lg = const_v[pl.ds(2 * E * L, L)]
        for e in range(E):
            fe = (t_v[pl.ds(e * L, L)]
                  - n0f * const_v[pl.ds(e * L, L)]) * rdenom
            lg = lg + fe * const_v[pl.ds(E * L + e * L, L)]
            plsc.store_scatter(feat_v, [lane * E + e], fe)
        logit_v[pl.ds(p * L, L)] = lg
        pltpu.sync_copy(
            feat_v, feat_hbm.at[pl.ds(pair_row0 * E, 2 * G * E)])

    load_block(0, idx_a, rows_a, sem_a)

    def pair(p, _):
        load_block(2 * p + 1, idx_b, rows_b, sem_b)
        drain(rows_a, sem_a)
        dim_phase(idx_a, rows_a, 0)

        @pl.when(p < NPAIR - 1)
        def _():
            load_block(2 * p + 2, idx_a, rows_a, sem_a)

        drain(rows_b, sem_b)
        dim_phase(idx_b, rows_b, 1)
        row_phase(p)
        return 0

    lax.fori_loop(0, NPAIR, pair, 0)
    pltpu.sync_copy(logit_v,
                    logits_hbm.at[pl.ds(base_row, ROWS_PER_W)])


@jax.jit
def _run(x_flat, embed_table, const):
    mesh = plsc.VectorSubcoreMesh(core_axis_name="c", subcore_axis_name="s")
    fn = pl.kernel(
        _sc_body,
        mesh=mesh,
        compiler_params=pltpu.CompilerParams(
            needs_layout_passes=False, use_tc_tiling_on_sc=False),
        out_type=[
            jax.ShapeDtypeStruct((B,), jnp.float32),
            jax.ShapeDtypeStruct((B * E,), jnp.float32),
        ],
        scratch_types=[
            pltpu.VMEM((IDX_PER_G + L,), jnp.int32),
            pltpu.VMEM((IDX_PER_G + L,), jnp.int32),
            pltpu.VMEM((IDX_PER_G, E), jnp.float32),
            pltpu.VMEM((IDX_PER_G, E), jnp.float32),
            pltpu.VMEM((2 * E * L + L,), jnp.float32),
            pltpu.VMEM((E * L,), jnp.float32),
            pltpu.VMEM((L * L,), jnp.float32),
            pltpu.VMEM((2 * G * E,), jnp.float32),
            pltpu.VMEM((ROWS_PER_W,), jnp.float32),
            pltpu.SemaphoreType.DMA,
            pltpu.SemaphoreType.DMA,
        ],
    )
    logits_flat, feat_flat = fn(x_flat, embed_table, const)
    return logits_flat.reshape(B, 1), feat_flat.reshape(B, E)


def kernel(x, embed_table, W, b):
    x_flat = jnp.asarray(x).astype(jnp.int32).reshape(-1)
    t0b = jnp.broadcast_to(embed_table[0][:, None], (E, L)).reshape(-1)
    wb = jnp.broadcast_to(
        W.astype(jnp.float32).reshape(E, 1), (E, L)).reshape(-1)
    bb = jnp.broadcast_to(b.astype(jnp.float32), (L,))
    const = jnp.concatenate([t0b, wb, bb])
    return _run(x_flat, embed_table, const)
